# baseline (speedup 1.0000x reference)
import jax
import jax.numpy as jnp
from jax import lax
from jax.experimental import pallas as pl
from jax.experimental.pallas import tpu as pltpu
from jax.experimental.pallas import tpu_sc as plsc

_NUM_EMB = 28
_EMBED_DIM = 128
_N_NODES = 100000

_NC = 2
_NS = 16
_NW = _NC * _NS

_CHUNK = 128
_K = 25
_ROWS_PER_W = _K * _CHUNK
_REM = _N_NODES - (_NW - 1) * _ROWS_PER_W
_TAIL = _N_NODES % _CHUNK
_NBUF = 6


def _gather_body(idx_hbm, table_hbm, out_hbm, idx_v, table_sh, bufs,
                 gsems, ssems):
    wid = lax.axis_index("s") * _NC + lax.axis_index("c")
    base = wid * _ROWS_PER_W

    whole = base + _ROWS_PER_W <= _N_NODES

    @pl.when(whole)
    def _():
        pltpu.async_copy(
            idx_hbm.at[pl.ds(base, _ROWS_PER_W)], idx_v, gsems[0]
        )

    @pl.when(jnp.logical_not(whole))
    def _():
        pltpu.async_copy(
            idx_hbm.at[pl.ds(base, _REM)], idx_v.at[pl.ds(0, _REM)],
            gsems[0],
        )

    @pl.when(lax.axis_index("s") == 0)
    def _():
        pltpu.sync_copy(table_hbm, table_sh)

    @pl.when(whole)
    def _():
        pltpu.make_async_copy(
            idx_hbm.at[pl.ds(base, _ROWS_PER_W)], idx_v, gsems[0]
        ).wait()

    @pl.when(jnp.logical_not(whole))
    def _():
        pltpu.make_async_copy(
            idx_hbm.at[pl.ds(base, _REM)], idx_v.at[pl.ds(0, _REM)],
            gsems[0],
        ).wait()

    plsc.subcore_barrier()

    def row0(k):
        return base + k * _CHUNK

    def active(k):
        return row0(k) < _N_NODES

    def full(k):
        return row0(k) + _CHUNK <= _N_NODES

    def partial(k):
        return jnp.logical_and(active(k), jnp.logical_not(full(k)))

    def start_gather(k):
        b = k % _NBUF

        @pl.when(full(k))
        def _():
            pltpu.async_copy(
                table_sh.at[idx_v.at[pl.ds(k * _CHUNK, _CHUNK)]],
                bufs[b], gsems[b],
            )

        @pl.when(partial(k))
        def _():
            pltpu.async_copy(
                table_sh.at[idx_v.at[pl.ds(k * _CHUNK, _TAIL)]],
                bufs[b].at[pl.ds(0, _TAIL)], gsems[b],
            )

    def wait_gather(k):
        b = k % _NBUF

        @pl.when(full(k))
        def _():
            pltpu.make_async_copy(
                table_sh.at[idx_v.at[pl.ds(k * _CHUNK, _CHUNK)]],
                bufs[b], gsems[b],
            ).wait()

        @pl.when(partial(k))
        def _():
            pltpu.make_async_copy(
                table_sh.at[idx_v.at[pl.ds(k * _CHUNK, _TAIL)]],
                bufs[b].at[pl.ds(0, _TAIL)], gsems[b],
            ).wait()

    def start_store(k):
        b = k % _NBUF

        @pl.when(full(k))
        def _():
            pltpu.async_copy(
                bufs[b], out_hbm.at[pl.ds(row0(k), _CHUNK)], ssems[b]
            )

        @pl.when(partial(k))
        def _():
            pltpu.async_copy(
                bufs[b].at[pl.ds(0, _TAIL)],
                out_hbm.at[pl.ds(row0(k), _TAIL)],
                ssems[b],
            )

    def wait_store(k):
        b = k % _NBUF

        @pl.when(full(k))
        def _():
            pltpu.make_async_copy(
                bufs[b], out_hbm.at[pl.ds(row0(k), _CHUNK)], ssems[b]
            ).wait()

        @pl.when(partial(k))
        def _():
            pltpu.make_async_copy(
                bufs[b].at[pl.ds(0, _TAIL)],
                out_hbm.at[pl.ds(row0(k), _TAIL)],
                ssems[b],
            ).wait()

    for k in range(_K):
        if k >= _NBUF:
            wait_store(k - _NBUF)
        start_gather(k)
        if k >= 1:
            wait_gather(k - 1)
            start_store(k - 1)
    wait_gather(_K - 1)
    start_store(_K - 1)
    for k in range(_K - _NBUF, _K):
        wait_store(k)


_gather = pl.kernel(
    _gather_body,
    out_type=jax.ShapeDtypeStruct((_N_NODES, _EMBED_DIM), jnp.float32),
    mesh=plsc.VectorSubcoreMesh(core_axis_name="c", subcore_axis_name="s"),
    scratch_types=[
        pltpu.VMEM((_ROWS_PER_W,), jnp.int32),
        pltpu.MemorySpace.VMEM_SHARED((_NUM_EMB, _EMBED_DIM), jnp.float32),
        [pltpu.VMEM((_CHUNK, _EMBED_DIM), jnp.float32) for _ in range(_NBUF)],
        [pltpu.SemaphoreType.DMA for _ in range(_NBUF)],
        [pltpu.SemaphoreType.DMA for _ in range(_NBUF)],
    ],
)


def kernel(tensor, table):
    return _gather(tensor, table)


# speedup vs baseline: 1.0275x; 1.0275x over previous
import jax
import jax.numpy as jnp
from jax import lax
from jax.experimental import pallas as pl
from jax.experimental.pallas import tpu as pltpu
from jax.experimental.pallas import tpu_sc as plsc

_NUM_EMB = 28
_EMBED_DIM = 128
_N_NODES = 100000

_NC = 2
_NS = 16
_NW = _NC * _NS

_CHUNK = 128
_K = 25
_ROWS_PER_W = _K * _CHUNK
_REM = _N_NODES - (_NW - 1) * _ROWS_PER_W
_TAIL = _N_NODES % _CHUNK
_NBUF = 7


def _gather_body(idx_hbm, table_hbm, out_hbm, idx_v, table_sh, bufs,
                 gsems, ssems):
    wid = lax.axis_index("s") * _NC + lax.axis_index("c")
    base = wid * _ROWS_PER_W

    whole = base + _ROWS_PER_W <= _N_NODES

    @pl.when(whole)
    def _():
        pltpu.async_copy(
            idx_hbm.at[pl.ds(base, _ROWS_PER_W)], idx_v, gsems[0]
        )

    @pl.when(jnp.logical_not(whole))
    def _():
        pltpu.async_copy(
            idx_hbm.at[pl.ds(base, _REM)], idx_v.at[pl.ds(0, _REM)],
            gsems[0],
        )

    @pl.when(lax.axis_index("s") == 0)
    def _():
        pltpu.sync_copy(table_hbm, table_sh)

    @pl.when(whole)
    def _():
        pltpu.make_async_copy(
            idx_hbm.at[pl.ds(base, _ROWS_PER_W)], idx_v, gsems[0]
        ).wait()

    @pl.when(jnp.logical_not(whole))
    def _():
        pltpu.make_async_copy(
            idx_hbm.at[pl.ds(base, _REM)], idx_v.at[pl.ds(0, _REM)],
            gsems[0],
        ).wait()

    plsc.subcore_barrier()

    def row0(k):
        return base + k * _CHUNK

    def active(k):
        return row0(k) < _N_NODES

    def full(k):
        return row0(k) + _CHUNK <= _N_NODES

    def partial(k):
        return jnp.logical_and(active(k), jnp.logical_not(full(k)))

    def start_gather(k):
        b = k % _NBUF

        @pl.when(full(k))
        def _():
            pltpu.async_copy(
                table_sh.at[idx_v.at[pl.ds(k * _CHUNK, _CHUNK)]],
                bufs[b], gsems[b],
            )

        @pl.when(partial(k))
        def _():
            pltpu.async_copy(
                table_sh.at[idx_v.at[pl.ds(k * _CHUNK, _TAIL)]],
                bufs[b].at[pl.ds(0, _TAIL)], gsems[b],
            )

    def wait_gather(k):
        b = k % _NBUF

        @pl.when(full(k))
        def _():
            pltpu.make_async_copy(
                table_sh.at[idx_v.at[pl.ds(k * _CHUNK, _CHUNK)]],
                bufs[b], gsems[b],
            ).wait()

        @pl.when(partial(k))
        def _():
            pltpu.make_async_copy(
                table_sh.at[idx_v.at[pl.ds(k * _CHUNK, _TAIL)]],
                bufs[b].at[pl.ds(0, _TAIL)], gsems[b],
            ).wait()

    def start_store(k):
        b = k % _NBUF

        @pl.when(full(k))
        def _():
            pltpu.async_copy(
                bufs[b], out_hbm.at[pl.ds(row0(k), _CHUNK)], ssems[b]
            )

        @pl.when(partial(k))
        def _():
            pltpu.async_copy(
                bufs[b].at[pl.ds(0, _TAIL)],
                out_hbm.at[pl.ds(row0(k), _TAIL)],
                ssems[b],
            )

    def wait_store(k):
        b = k % _NBUF

        @pl.when(full(k))
        def _():
            pltpu.make_async_copy(
                bufs[b], out_hbm.at[pl.ds(row0(k), _CHUNK)], ssems[b]
            ).wait()

        @pl.when(partial(k))
        def _():
            pltpu.make_async_copy(
                bufs[b].at[pl.ds(0, _TAIL)],
                out_hbm.at[pl.ds(row0(k), _TAIL)],
                ssems[b],
            ).wait()

    for k in range(_K):
        if k >= _NBUF:
            wait_store(k - _NBUF)
        start_gather(k)
        if k >= 2:
            wait_gather(k - 2)
            start_store(k - 2)
    for k in range(_K - 2, _K):
        wait_gather(k)
        start_store(k)
    for k in range(_K - _NBUF, _K):
        wait_store(k)


_gather = pl.kernel(
    _gather_body,
    out_type=jax.ShapeDtypeStruct((_N_NODES, _EMBED_DIM), jnp.float32),
    mesh=plsc.VectorSubcoreMesh(core_axis_name="c", subcore_axis_name="s"),
    scratch_types=[
        pltpu.VMEM((_ROWS_PER_W,), jnp.int32),
        pltpu.MemorySpace.VMEM_SHARED((_NUM_EMB, _EMBED_DIM), jnp.float32),
        [pltpu.VMEM((_CHUNK, _EMBED_DIM), jnp.float32) for _ in range(_NBUF)],
        [pltpu.SemaphoreType.DMA for _ in range(_NBUF)],
        [pltpu.SemaphoreType.DMA for _ in range(_NBUF)],
    ],
)


def kernel(tensor, table):
    return _gather(tensor, table)
